# baseline (speedup 1.0000x reference)
import functools

import jax
import jax.numpy as jnp
from jax import lax
from jax.experimental import pallas as pl
from jax.experimental.pallas import tpu as pltpu
from jax.experimental.pallas import tpu_sc as plsc

EMB_DIM = 32
NUM_CORES = 2
NUM_SUBCORES = 16
NUM_WORKERS = NUM_CORES * NUM_SUBCORES
CHUNK_B = 8


@functools.lru_cache(maxsize=None)
def _make_gather(batch: int, seq: int):
    rows_b = batch // NUM_WORKERS
    n_chunks = rows_b // CHUNK_B
    chunk = CHUNK_B * seq
    mesh = plsc.VectorSubcoreMesh(core_axis_name="c", subcore_axis_name="s")

    @functools.partial(
        pl.kernel,
        mesh=mesh,
        compiler_params=pltpu.CompilerParams(use_tc_tiling_on_sc=False),
        out_type=jax.ShapeDtypeStruct((batch, seq, EMB_DIM), jnp.float32),
        scratch_types=[
            pltpu.VMEM((n_chunks, chunk), jnp.int32),
            pltpu.VMEM((chunk, EMB_DIM), jnp.float32),
            pltpu.VMEM((chunk, EMB_DIM), jnp.float32),
            pltpu.SemaphoreType.DMA,
            pltpu.SemaphoreType.DMA,
            pltpu.SemaphoreType.DMA,
            pltpu.SemaphoreType.DMA,
        ],
    )
    def gather_kernel(idx_hbm, table_hbm, out_hbm, idx_v, rows0, rows1,
                      g0, g1, s0, s1):
        wid = lax.axis_index("s") * NUM_CORES + lax.axis_index("c")
        pltpu.sync_copy(idx_hbm.at[pl.ds(wid * n_chunks, n_chunks)], idx_v)

        bufs = (rows0, rows1)
        gsems = (g0, g1)
        ssems = (s0, s1)
        gh = [None] * n_chunks
        sh = [None] * n_chunks

        def store_chunk(i, buf, sem):
            b0 = wid * rows_b + i * CHUNK_B
            h = None
            for j in range(CHUNK_B):
                h = pltpu.async_copy(
                    buf.at[pl.ds(j * seq, seq)], out_hbm.at[b0 + j], sem)
            return h

        def wait_store(i):
            for _ in range(CHUNK_B):
                sh[i].wait()

        gh[0] = pltpu.async_copy(table_hbm.at[idx_v.at[0]], bufs[0], gsems[0])
        for i in range(n_chunks):
            cur = i % 2
            if i + 1 < n_chunks:
                nxt = (i + 1) % 2
                if i >= 1:
                    wait_store(i - 1)
                gh[i + 1] = pltpu.async_copy(
                    table_hbm.at[idx_v.at[i + 1]], bufs[nxt], gsems[nxt])
            gh[i].wait()
            sh[i] = store_chunk(i, bufs[cur], ssems[cur])
        if n_chunks >= 2:
            wait_store(n_chunks - 2)
        wait_store(n_chunks - 1)

    return gather_kernel


def kernel(sentence, table):
    batch, seq = sentence.shape
    chunk = CHUNK_B * seq
    idx = sentence.reshape(batch * seq // chunk, chunk).astype(jnp.int32)
    return _make_gather(batch, seq)(idx, table)
